# Initial kernel scaffold; baseline (speedup 1.0000x reference)
import jax
import jax.numpy as jnp
from jax.experimental import pallas as pl


def kernel(z, angle_idx, W1, b1, W2, b2):
    raise NotImplementedError("write your pallas kernel here")


# speedup vs baseline: 1.2959x; 1.2959x over previous
import functools

import jax
import jax.numpy as jnp
from jax.experimental import pallas as pl
from jax.experimental.pallas import tpu as pltpu

E = 8
D = 1024
H = 400
O = 1024
B = 4096
T = 256
NT = B // T
S = NT + E


def _mlp_body(tile_ref, exp_ref, lo_ref, hi_ref,
              z_ref, w1_ref, b1_ref, w2_ref, b2_ref, out_ref):
    s = pl.program_id(0)
    x = z_ref[...]
    h = jax.lax.dot_general(x, w1_ref[0], (((1,), (0,)), ((), ())),
                            preferred_element_type=jnp.float32)
    h = jnp.maximum(h + b1_ref[0], 0.0)
    y = jax.lax.dot_general(h, w2_ref[0], (((1,), (0,)), ((), ())),
                            preferred_element_type=jnp.float32)
    y = jax.nn.sigmoid(y + b2_ref[0])
    rid = jax.lax.broadcasted_iota(jnp.int32, (T, O), 0)
    mask = (rid >= lo_ref[s]) & (rid < hi_ref[s])
    out_ref[...] = jnp.where(mask, y, out_ref[...])


@functools.partial(jax.jit, static_argnames=())
def _grouped_mlp(z_sorted, W1, b1, W2, b2, tile_id, exp_id, lo, hi):
    grid_spec = pltpu.PrefetchScalarGridSpec(
        num_scalar_prefetch=4,
        grid=(S,),
        in_specs=[
            pl.BlockSpec((T, D), lambda s, t, e, lo, hi: (t[s], 0)),
            pl.BlockSpec((1, D, H), lambda s, t, e, lo, hi: (e[s], 0, 0)),
            pl.BlockSpec((1, 1, H), lambda s, t, e, lo, hi: (e[s], 0, 0)),
            pl.BlockSpec((1, H, O), lambda s, t, e, lo, hi: (e[s], 0, 0)),
            pl.BlockSpec((1, 1, O), lambda s, t, e, lo, hi: (e[s], 0, 0)),
        ],
        out_specs=pl.BlockSpec((T, O), lambda s, t, e, lo, hi: (t[s], 0)),
    )
    return pl.pallas_call(
        _mlp_body,
        grid_spec=grid_spec,
        out_shape=jax.ShapeDtypeStruct((B, O), jnp.float32),
    )(tile_id, exp_id, lo, hi, z_sorted, W1, b1, W2, b2)


def kernel(z, angle_idx, W1, b1, W2, b2):
    e32 = angle_idx.astype(jnp.int32)
    sort_idx = jnp.argsort(e32, stable=True).astype(jnp.int32)
    counts = jnp.sum(e32[None, :] == jnp.arange(E, dtype=jnp.int32)[:, None],
                     axis=1).astype(jnp.int32)
    csum = jnp.concatenate([jnp.zeros((1,), jnp.int32),
                            jnp.cumsum(counts)[:-1].astype(jnp.int32)])
    tile_starts = (jnp.arange(NT, dtype=jnp.int32) * T)
    starts = jnp.sort(jnp.concatenate([tile_starts, csum]))
    ends = jnp.concatenate([starts[1:], jnp.full((1,), B, jnp.int32)])
    tile_id = jnp.minimum(starts // T, NT - 1)
    exp_id = jnp.clip(jnp.searchsorted(csum, starts, side="right") - 1, 0, E - 1
                      ).astype(jnp.int32)
    lo = starts - tile_id * T
    hi = jnp.maximum(ends - tile_id * T, lo)

    z_sorted = jnp.take(z, sort_idx, axis=0)
    y_sorted = _grouped_mlp(z_sorted, W1, b1.reshape(E, 1, H),
                            W2, b2.reshape(E, 1, O), tile_id, exp_id, lo, hi)
    out = jnp.zeros((B, O), jnp.float32).at[sort_idx].set(y_sorted)
    return out
